# baseline (speedup 1.0000x reference)
import functools

import jax
import jax.numpy as jnp
from jax import lax
from jax.experimental import pallas as pl
from jax.experimental.pallas import tpu as pltpu
from jax.experimental.pallas import tpu_sc as plsc

N = 50000
E = 800000
H = 64

NC = 2
NS = 16
NW = NC * NS

CH = 128
GK = 8

G_CHUNKS = (2 * E) // CH
G_ITERS = 49
G_CHUNKS_PAD = NW * GK * G_ITERS
R_PAD = G_CHUNKS_PAD * CH
W_CHUNKS = G_CHUNKS_PAD // NW

S_CHUNKS = E // CH
SK = 4
S_MAIN_ITERS = 96
S_MAIN = S_MAIN_ITERS * SK
S_EXTRA_BASE = NS * S_MAIN
ZROWS = N // NS

BE = 2000
BN = 2000
GE = E // BE
GN = N // BN

_MESH = plsc.VectorSubcoreMesh(core_axis_name="c", subcore_axis_name="s")


def _make_gather(d, dtype):
    @functools.partial(
        pl.kernel,
        out_type=jax.ShapeDtypeStruct((R_PAD, d), dtype),
        mesh=_MESH,
        scratch_types=[
            pltpu.VMEM((GK, CH), jnp.int32),
            pltpu.VMEM((GK * CH, d), dtype),
            pltpu.SemaphoreType.DMA,
        ],
        compiler_params=pltpu.CompilerParams(use_tc_tiling_on_sc=False),
    )
    def gather_k(table_hbm, idx_hbm, out_hbm, idx_v, rows_v, sem):
        wid = lax.axis_index("s") * NC + lax.axis_index("c")
        base = wid * W_CHUNKS

        def step(i, carry):
            cid0 = base + i * GK
            pltpu.sync_copy(idx_hbm.at[pl.ds(cid0, GK)], idx_v)
            cps = [
                pltpu.async_copy(
                    table_hbm.at[idx_v.at[k]],
                    rows_v.at[pl.ds(k * CH, CH)],
                    sem,
                )
                for k in range(GK)
            ]
            for cp in cps:
                cp.wait()
            pltpu.sync_copy(rows_v, out_hbm.at[pl.ds(cid0 * CH, GK * CH)])
            return carry

        lax.fori_loop(0, G_ITERS, step, 0)

    return gather_k


_GATHER80 = _make_gather(80, jnp.float32)
_GATHER64 = _make_gather(64, jnp.float32)


@functools.partial(
    pl.kernel,
    out_type=jax.ShapeDtypeStruct((NC, N, H // 2), jnp.float32),
    mesh=_MESH,
    scratch_types=[
        pltpu.VMEM((SK, CH), jnp.int32),
        pltpu.VMEM((SK * CH, H // 2), jnp.float32),
        pltpu.VMEM_SHARED((N, H // 2), jnp.float32),
    ],
    compiler_params=pltpu.CompilerParams(use_tc_tiling_on_sc=False),
)
def _scatter_add(msg_hbm, idx_hbm, zero_hbm, out_hbm, idx_v, msg_v, shared):
    c = lax.axis_index("c")
    t = lax.axis_index("s")
    pltpu.sync_copy(zero_hbm, shared.at[pl.ds(t * ZROWS, ZROWS)])
    plsc.subcore_barrier()

    def step(i, carry):
        cid0 = t * S_MAIN + i * SK
        pltpu.sync_copy(idx_hbm.at[pl.ds(cid0, SK)], idx_v)
        pltpu.sync_copy(msg_hbm.at[c, pl.ds(cid0 * CH, SK * CH)], msg_v)
        for k in range(SK):
            pltpu.sync_copy(
                msg_v.at[pl.ds(k * CH, CH)], shared.at[idx_v.at[k]], add=True
            )
        return carry

    lax.fori_loop(0, S_MAIN_ITERS, step, 0)

    extra_cnt = jnp.where(t < 10, 7, 6)
    extra_base = S_EXTRA_BASE + t * 6 + jnp.minimum(t, 10)

    def estep(j, carry):
        cid = extra_base + j
        pltpu.sync_copy(idx_hbm.at[pl.ds(cid, 1)], idx_v.at[pl.ds(0, 1)])
        pltpu.sync_copy(msg_hbm.at[c, pl.ds(cid * CH, CH)], msg_v.at[pl.ds(0, CH)])
        pltpu.sync_copy(msg_v.at[pl.ds(0, CH)], shared.at[idx_v.at[0]], add=True)
        return carry

    lax.fori_loop(0, extra_cnt, estep, 0)
    plsc.subcore_barrier()
    pltpu.sync_copy(
        shared.at[pl.ds(t * ZROWS, ZROWS)], out_hbm.at[c, pl.ds(t * ZROWS, ZROWS)]
    )


def _mm(a, b):
    return lax.dot_general(
        a, b, (((1,), (0,)), ((), ())),
        preferred_element_type=jnp.float32,
    )


def _ln(y, g, beta):
    mu = jnp.mean(y, axis=-1, keepdims=True)
    var = jnp.mean((y - mu) ** 2, axis=-1, keepdims=True)
    return (y - mu) / jnp.sqrt(var + 1e-5) * g + beta


def _full(x):
    return pl.BlockSpec(x.shape, lambda i: tuple(0 for _ in x.shape))


def _mlp_body(x_ref, w0_ref, b0_ref, w1_ref, b1_ref, g_ref, beta_ref, o_ref):
    h = jnp.maximum(_mm(x_ref[...], w0_ref[...]) + b0_ref[...], 0.0)
    y = _mm(h, w1_ref[...]) + b1_ref[...]
    o_ref[...] = _ln(y, g_ref[...], beta_ref[...])


def _mlp_call(x, w0t, b0, w1t, b1, g, beta, bm):
    n = x.shape[0]
    ws = (w0t, b0, w1t, b1, g, beta)
    return pl.pallas_call(
        _mlp_body,
        grid=(n // bm,),
        in_specs=[pl.BlockSpec((bm, x.shape[1]), lambda i: (i, 0))]
        + [_full(w) for w in ws],
        out_specs=pl.BlockSpec((bm, H), lambda i: (i, 0)),
        out_shape=jax.ShapeDtypeStruct((n, H), jnp.float32),
    )(x, *ws)


def _edge_enc_body(nfs_ref, nfr_ref, w0_ref, b0_ref, w1_ref, b1_ref, g_ref,
                   beta_ref, o_ref):
    d = nfs_ref[...] - nfr_ref[...]
    dist = jnp.sqrt(d[:, 0:1] ** 2 + d[:, 1:2] ** 2 + d[:, 2:3] ** 2)
    dist_w = jnp.sqrt(d[:, 3:4] ** 2 + d[:, 4:5] ** 2 + d[:, 5:6] ** 2)
    e = jnp.concatenate(
        [d[:, 0:3], dist, d[:, 3:6], dist_w, d[:, 6:7],
         jnp.zeros_like(d[:, 0:7])], axis=-1)
    h = jnp.maximum(_mm(e, w0_ref[...]) + b0_ref[...], 0.0)
    y = _mm(h, w1_ref[...]) + b1_ref[...]
    o_ref[...] = _ln(y, g_ref[...], beta_ref[...])


def _edge_enc_call(nf_g, w0t, b0, w1t, b1, g, beta):
    ws = (w0t, b0, w1t, b1, g, beta)
    return pl.pallas_call(
        _edge_enc_body,
        grid=(GE,),
        in_specs=[
            pl.BlockSpec((BE, 16), lambda i: (i, 0)),
            pl.BlockSpec((BE, 16), lambda i: (i + GE, 0)),
        ]
        + [_full(w) for w in ws],
        out_specs=pl.BlockSpec((BE, H), lambda i: (i, 0)),
        out_shape=jax.ShapeDtypeStruct((E, H), jnp.float32),
    )(nf_g, nf_g, *ws)


def _pe_math(xs, xr, eh, w0, b0, w1, b1, g_, beta):
    h1 = jnp.maximum(_mm(jnp.concatenate([xr, xs, eh], axis=-1), w0) + b0, 0.0)
    m = _ln(_mm(h1, w1) + b1, g_, beta)
    h2 = jnp.maximum(_mm(jnp.concatenate([xs, xr, eh], axis=-1), w0) + b0, 0.0)
    ne = _ln(_mm(h2, w1) + b1, g_, beta)
    return m, ne


def _ee_math(nfs, nfr, w0, b0, w1, b1, g_, beta):
    d = nfs - nfr
    dist = jnp.sqrt(d[:, 0:1] ** 2 + d[:, 1:2] ** 2 + d[:, 2:3] ** 2)
    dist_w = jnp.sqrt(d[:, 3:4] ** 2 + d[:, 4:5] ** 2 + d[:, 5:6] ** 2)
    e = jnp.concatenate(
        [d[:, 0:3], dist, d[:, 3:6], dist_w, d[:, 6:7],
         jnp.zeros_like(d[:, 0:7])], axis=-1)
    h = jnp.maximum(_mm(e, w0) + b0, 0.0)
    y = _mm(h, w1) + b1
    return _ln(y, g_, beta)


def _pe_body(xgs_ref, xgr_ref, eh_ref, w0_ref, b0_ref,
             w1_ref, b1_ref, g_ref, beta_ref, msg_ref, eo_ref):
    eh = eh_ref[...]
    m, ne = _pe_math(xgs_ref[...], xgr_ref[...], eh, w0_ref[...], b0_ref[...],
                     w1_ref[...], b1_ref[...], g_ref[...], beta_ref[...])
    eo_ref[...] = eh + ne
    msg_ref[0] = m[:, 0:32]
    msg_ref[1] = m[:, 32:64]


def _pe1_body(gs_ref, gr_ref,
              ew0_ref, eb0_ref, ew1_ref, eb1_ref, eg_ref, ebeta_ref,
              w0_ref, b0_ref, w1_ref, b1_ref, g_ref, beta_ref,
              msg_ref, eo_ref):
    gs = gs_ref[...]
    gr = gr_ref[...]
    eh = _ee_math(gs[:, 64:80], gr[:, 64:80], ew0_ref[...], eb0_ref[...],
                  ew1_ref[...], eb1_ref[...], eg_ref[...], ebeta_ref[...])
    m, ne = _pe_math(gs[:, 0:64], gr[:, 0:64], eh, w0_ref[...], b0_ref[...],
                     w1_ref[...], b1_ref[...], g_ref[...], beta_ref[...])
    eo_ref[...] = eh + ne
    msg_ref[0] = m[:, 0:32]
    msg_ref[1] = m[:, 32:64]


def _pe1_call(xnf_g, ee_ws, pe_ws):
    ws = tuple(ee_ws) + tuple(pe_ws)
    return pl.pallas_call(
        _pe1_body,
        grid=(GE,),
        in_specs=[
            pl.BlockSpec((BE, 80), lambda i: (i, 0)),
            pl.BlockSpec((BE, 80), lambda i: (i + GE, 0)),
        ]
        + [_full(w) for w in ws],
        out_specs=[
            pl.BlockSpec((NC, BE, H // 2), lambda i: (0, i, 0)),
            pl.BlockSpec((BE, H), lambda i: (i, 0)),
        ],
        out_shape=[
            jax.ShapeDtypeStruct((NC, E, H // 2), jnp.float32),
            jax.ShapeDtypeStruct((E, H), jnp.float32),
        ],
    )(xnf_g, xnf_g, *ws)


def _pe_call(xg, eh, w0t, b0, w1t, b1, g, beta):
    ws = (w0t, b0, w1t, b1, g, beta)
    return pl.pallas_call(
        _pe_body,
        grid=(GE,),
        in_specs=[
            pl.BlockSpec((BE, H), lambda i: (i, 0)),
            pl.BlockSpec((BE, H), lambda i: (i + GE, 0)),
            pl.BlockSpec((BE, H), lambda i: (i, 0)),
        ]
        + [_full(w) for w in ws],
        out_specs=[
            pl.BlockSpec((NC, BE, H // 2), lambda i: (0, i, 0)),
            pl.BlockSpec((BE, H), lambda i: (i, 0)),
        ],
        out_shape=[
            jax.ShapeDtypeStruct((NC, E, H // 2), jnp.float32),
            jax.ShapeDtypeStruct((E, H), jnp.float32),
        ],
    )(xg, xg, eh, *ws)


def _pn_body(a0_ref, a1_ref, xh_ref, w0_ref, b0_ref, w1_ref, b1_ref,
             g_ref, beta_ref, o_ref):
    xh = xh_ref[...]
    cat = jnp.concatenate([a0_ref[0], a1_ref[0], xh], axis=-1)
    h = jnp.maximum(_mm(cat, w0_ref[...]) + b0_ref[...], 0.0)
    y = _mm(h, w1_ref[...]) + b1_ref[...]
    o_ref[...] = xh + _ln(y, g_ref[...], beta_ref[...])


def _pn_call(aggr2, xh, w0t, b0, w1t, b1, g, beta):
    ws = (w0t, b0, w1t, b1, g, beta)
    return pl.pallas_call(
        _pn_body,
        grid=(GN,),
        in_specs=[
            pl.BlockSpec((1, BN, H // 2), lambda i: (0, i, 0)),
            pl.BlockSpec((1, BN, H // 2), lambda i: (1, i, 0)),
            pl.BlockSpec((BN, H), lambda i: (i, 0)),
        ]
        + [_full(w) for w in ws],
        out_specs=pl.BlockSpec((BN, H), lambda i: (i, 0)),
        out_shape=jax.ShapeDtypeStruct((N, H), jnp.float32),
    )(aggr2, aggr2, xh, *ws)


def _dec_body(xh_ref, w1_ref, b1_ref, w2_ref, b2_ref, o_ref):
    h = _mm(xh_ref[...], w1_ref[...]) + b1_ref[...]
    h = h * jax.nn.sigmoid(h)
    o_ref[...] = _mm(h, w2_ref[...]) + b2_ref[...]


def _dec_call(xh, w1t, b1, w2t, b2):
    ws = (w1t, b1, w2t, b2)
    return pl.pallas_call(
        _dec_body,
        grid=(GN,),
        in_specs=[pl.BlockSpec((BN, H), lambda i: (i, 0))]
        + [_full(w) for w in ws],
        out_specs=pl.BlockSpec((BN, 8), lambda i: (i, 0)),
        out_shape=jax.ShapeDtypeStruct((N, 8), jnp.float32),
    )(xh, *ws)


def kernel(world_pos, mesh_pos, phi, swelling_phi, swelling_phi_rate,
           node_type, mat_param, params, edge_index):
    f32 = jnp.float32
    mat = jnp.broadcast_to(mat_param[None, :].astype(f32), (N, 4))
    x = jnp.concatenate(
        [phi, swelling_phi, swelling_phi_rate, node_type, mat], axis=-1
    )
    nf = jnp.concatenate(
        [mesh_pos, world_pos, phi, jnp.zeros((N, 9), f32)], axis=-1
    )

    ei = edge_index.astype(jnp.int32)
    idx_flat = jnp.concatenate(
        [ei[0], ei[1], jnp.zeros((R_PAD - 2 * E,), jnp.int32)]
    ).reshape(G_CHUNKS_PAD, CH)
    sidx = ei[1].reshape(S_CHUNKS, CH)
    zrows = jnp.zeros((ZROWS, H // 2), f32)

    ne = params['node_enc']
    x_h = _mlp_call(x, ne['W0'].T, ne['b0'][None], ne['W1'].T, ne['b1'][None],
                    ne['g'][None], ne['beta'][None], BN)

    ee = params['edge_enc']
    w0e9 = jnp.concatenate([ee['W0'].T, jnp.zeros((7, H), f32)], axis=0)
    ee_ws = (w0e9, ee['b0'][None], ee['W1'].T, ee['b1'][None],
             ee['g'][None], ee['beta'][None])

    e_h = None
    msg2 = None
    for si, p in enumerate(params['procs']):
        em = p['edge_mlp']
        nm = p['node_mlp']
        pe_ws = (em['W0'].T, em['b0'][None], em['W1'].T, em['b1'][None],
                 em['g'][None], em['beta'][None])
        if si == 0:
            xnf_g = _GATHER80(jnp.concatenate([x_h, nf], axis=1), idx_flat)
            msg2, e_h = _pe1_call(xnf_g, ee_ws, pe_ws)
        else:
            xg = _GATHER64(x_h, idx_flat)
            msg2, e_h = _pe_call(xg, e_h, *pe_ws)
        aggr2 = _scatter_add(msg2, sidx, zrows)
        x_h = _pn_call(
            aggr2, x_h, nm['W0'].T, nm['b0'][None], nm['W1'].T, nm['b1'][None],
            nm['g'][None], nm['beta'][None],
        )

    d = params['dec']
    w2p = jnp.concatenate([d['W2'].T, jnp.zeros((H // 2, 5), f32)], axis=1)
    b2p = jnp.concatenate([d['b2'], jnp.zeros((5,), f32)])[None]
    dec = _dec_call(x_h, d['W1'].T, d['b1'][None], w2p, b2p)
    return dec[:, :3].reshape(1, N, 3)


# speedup vs baseline: 1.1712x; 1.1712x over previous
import functools

import jax
import jax.numpy as jnp
from jax import lax
from jax.experimental import pallas as pl
from jax.experimental.pallas import tpu as pltpu
from jax.experimental.pallas import tpu_sc as plsc

N = 50000
E = 800000
H = 64

NC = 2
NS = 16
NW = NC * NS

CH = 128
GK = 8

G_CHUNKS = (2 * E) // CH
G_ITERS = 49
G_CHUNKS_PAD = NW * GK * G_ITERS
R_PAD = G_CHUNKS_PAD * CH
W_CHUNKS = G_CHUNKS_PAD // NW

S_CHUNKS = E // CH
SK = 4
S_MAIN_ITERS = 96
S_MAIN = S_MAIN_ITERS * SK
S_EXTRA_BASE = NS * S_MAIN
ZROWS = N // NS

BE = 8000
BN = 2000
GE = E // BE
GN = N // BN

_MESH = plsc.VectorSubcoreMesh(core_axis_name="c", subcore_axis_name="s")


def _make_gather(d, dtype):
    @functools.partial(
        pl.kernel,
        out_type=jax.ShapeDtypeStruct((R_PAD, d), dtype),
        mesh=_MESH,
        scratch_types=[
            pltpu.VMEM((GK, CH), jnp.int32),
            pltpu.VMEM((GK * CH, d), dtype),
            pltpu.SemaphoreType.DMA,
        ],
        compiler_params=pltpu.CompilerParams(use_tc_tiling_on_sc=False),
    )
    def gather_k(table_hbm, idx_hbm, out_hbm, idx_v, rows_v, sem):
        wid = lax.axis_index("s") * NC + lax.axis_index("c")
        base = wid * W_CHUNKS

        def step(i, carry):
            cid0 = base + i * GK
            pltpu.sync_copy(idx_hbm.at[pl.ds(cid0, GK)], idx_v)
            cps = [
                pltpu.async_copy(
                    table_hbm.at[idx_v.at[k]],
                    rows_v.at[pl.ds(k * CH, CH)],
                    sem,
                )
                for k in range(GK)
            ]
            for cp in cps:
                cp.wait()
            pltpu.sync_copy(rows_v, out_hbm.at[pl.ds(cid0 * CH, GK * CH)])
            return carry

        lax.fori_loop(0, G_ITERS, step, 0)

    return gather_k


_GATHER80 = _make_gather(80, jnp.float32)
_GATHER64 = _make_gather(64, jnp.float32)


@functools.partial(
    pl.kernel,
    out_type=jax.ShapeDtypeStruct((NC, N, H // 2), jnp.float32),
    mesh=_MESH,
    scratch_types=[
        pltpu.VMEM((SK, CH), jnp.int32),
        pltpu.VMEM((SK * CH, H // 2), jnp.float32),
        pltpu.VMEM_SHARED((N, H // 2), jnp.float32),
    ],
    compiler_params=pltpu.CompilerParams(use_tc_tiling_on_sc=False),
)
def _scatter_add(msg_hbm, idx_hbm, zero_hbm, out_hbm, idx_v, msg_v, shared):
    c = lax.axis_index("c")
    t = lax.axis_index("s")
    pltpu.sync_copy(zero_hbm, shared.at[pl.ds(t * ZROWS, ZROWS)])
    plsc.subcore_barrier()

    def step(i, carry):
        cid0 = t * S_MAIN + i * SK
        pltpu.sync_copy(idx_hbm.at[pl.ds(cid0, SK)], idx_v)
        pltpu.sync_copy(msg_hbm.at[c, pl.ds(cid0 * CH, SK * CH)], msg_v)
        for k in range(SK):
            pltpu.sync_copy(
                msg_v.at[pl.ds(k * CH, CH)], shared.at[idx_v.at[k]], add=True
            )
        return carry

    lax.fori_loop(0, S_MAIN_ITERS, step, 0)

    extra_cnt = jnp.where(t < 10, 7, 6)
    extra_base = S_EXTRA_BASE + t * 6 + jnp.minimum(t, 10)

    def estep(j, carry):
        cid = extra_base + j
        pltpu.sync_copy(idx_hbm.at[pl.ds(cid, 1)], idx_v.at[pl.ds(0, 1)])
        pltpu.sync_copy(msg_hbm.at[c, pl.ds(cid * CH, CH)], msg_v.at[pl.ds(0, CH)])
        pltpu.sync_copy(msg_v.at[pl.ds(0, CH)], shared.at[idx_v.at[0]], add=True)
        return carry

    lax.fori_loop(0, extra_cnt, estep, 0)
    plsc.subcore_barrier()
    pltpu.sync_copy(
        shared.at[pl.ds(t * ZROWS, ZROWS)], out_hbm.at[c, pl.ds(t * ZROWS, ZROWS)]
    )


def _mm(a, b):
    return lax.dot_general(
        a, b, (((1,), (0,)), ((), ())),
        preferred_element_type=jnp.float32,
    )


def _ln(y, g, beta):
    mu = jnp.mean(y, axis=-1, keepdims=True)
    var = jnp.mean((y - mu) ** 2, axis=-1, keepdims=True)
    return (y - mu) / jnp.sqrt(var + 1e-5) * g + beta


def _full(x):
    return pl.BlockSpec(x.shape, lambda i: tuple(0 for _ in x.shape))


def _mlp_body(x_ref, w0_ref, b0_ref, w1_ref, b1_ref, g_ref, beta_ref, o_ref):
    h = jnp.maximum(_mm(x_ref[...], w0_ref[...]) + b0_ref[...], 0.0)
    y = _mm(h, w1_ref[...]) + b1_ref[...]
    o_ref[...] = _ln(y, g_ref[...], beta_ref[...])


def _mlp_call(x, w0t, b0, w1t, b1, g, beta, bm):
    n = x.shape[0]
    ws = (w0t, b0, w1t, b1, g, beta)
    return pl.pallas_call(
        _mlp_body,
        grid=(n // bm,),
        in_specs=[pl.BlockSpec((bm, x.shape[1]), lambda i: (i, 0))]
        + [_full(w) for w in ws],
        out_specs=pl.BlockSpec((bm, H), lambda i: (i, 0)),
        out_shape=jax.ShapeDtypeStruct((n, H), jnp.float32),
    )(x, *ws)


def _edge_enc_body(nfs_ref, nfr_ref, w0_ref, b0_ref, w1_ref, b1_ref, g_ref,
                   beta_ref, o_ref):
    d = nfs_ref[...] - nfr_ref[...]
    dist = jnp.sqrt(d[:, 0:1] ** 2 + d[:, 1:2] ** 2 + d[:, 2:3] ** 2)
    dist_w = jnp.sqrt(d[:, 3:4] ** 2 + d[:, 4:5] ** 2 + d[:, 5:6] ** 2)
    e = jnp.concatenate(
        [d[:, 0:3], dist, d[:, 3:6], dist_w, d[:, 6:7],
         jnp.zeros_like(d[:, 0:7])], axis=-1)
    h = jnp.maximum(_mm(e, w0_ref[...]) + b0_ref[...], 0.0)
    y = _mm(h, w1_ref[...]) + b1_ref[...]
    o_ref[...] = _ln(y, g_ref[...], beta_ref[...])


def _edge_enc_call(nf_g, w0t, b0, w1t, b1, g, beta):
    ws = (w0t, b0, w1t, b1, g, beta)
    return pl.pallas_call(
        _edge_enc_body,
        grid=(GE,),
        in_specs=[
            pl.BlockSpec((BE, 16), lambda i: (i, 0)),
            pl.BlockSpec((BE, 16), lambda i: (i + GE, 0)),
        ]
        + [_full(w) for w in ws],
        out_specs=pl.BlockSpec((BE, H), lambda i: (i, 0)),
        out_shape=jax.ShapeDtypeStruct((E, H), jnp.float32),
    )(nf_g, nf_g, *ws)


def _pe_math(xs, xr, eh, w0, b0, w1, b1, g_, beta):
    h1 = jnp.maximum(_mm(jnp.concatenate([xr, xs, eh], axis=-1), w0) + b0, 0.0)
    m = _ln(_mm(h1, w1) + b1, g_, beta)
    h2 = jnp.maximum(_mm(jnp.concatenate([xs, xr, eh], axis=-1), w0) + b0, 0.0)
    ne = _ln(_mm(h2, w1) + b1, g_, beta)
    return m, ne


def _ee_math(nfs, nfr, w0p, w3, w7, b0, w1, b1, g_, beta):
    d = nfs - nfr
    dist = jnp.sqrt(d[:, 0:1] ** 2 + d[:, 1:2] ** 2 + d[:, 2:3] ** 2)
    dist_w = jnp.sqrt(d[:, 3:4] ** 2 + d[:, 4:5] ** 2 + d[:, 5:6] ** 2)
    db = dist.astype(jnp.bfloat16).astype(jnp.float32)
    dwb = dist_w.astype(jnp.bfloat16).astype(jnp.float32)
    h = jnp.maximum(_mm(d, w0p) + db * w3 + dwb * w7 + b0, 0.0)
    y = _mm(h, w1) + b1
    return _ln(y, g_, beta)


def _pe_body(xgs_ref, xgr_ref, eh_ref, w0_ref, b0_ref,
             w1_ref, b1_ref, g_ref, beta_ref, msg_ref, eo_ref):
    eh = eh_ref[...]
    m, ne = _pe_math(xgs_ref[...], xgr_ref[...], eh, w0_ref[...], b0_ref[...],
                     w1_ref[...], b1_ref[...], g_ref[...], beta_ref[...])
    eo_ref[...] = eh + ne
    msg_ref[0] = m[:, 0:32]
    msg_ref[1] = m[:, 32:64]


def _pe1_body(gs_ref, gr_ref,
              ew0_ref, ew3_ref, ew7_ref, eb0_ref, ew1_ref, eb1_ref, eg_ref,
              ebeta_ref,
              w0_ref, b0_ref, w1_ref, b1_ref, g_ref, beta_ref,
              msg_ref, eo_ref):
    gs = gs_ref[...]
    gr = gr_ref[...]
    eh = _ee_math(gs[:, 64:80], gr[:, 64:80], ew0_ref[...], ew3_ref[...],
                  ew7_ref[...], eb0_ref[...], ew1_ref[...], eb1_ref[...],
                  eg_ref[...], ebeta_ref[...])
    m, ne = _pe_math(gs[:, 0:64], gr[:, 0:64], eh, w0_ref[...], b0_ref[...],
                     w1_ref[...], b1_ref[...], g_ref[...], beta_ref[...])
    eo_ref[...] = eh + ne
    msg_ref[0] = m[:, 0:32]
    msg_ref[1] = m[:, 32:64]


def _pe1_call(xnf_g, ee_ws, pe_ws):
    ws = tuple(ee_ws) + tuple(pe_ws)
    return pl.pallas_call(
        _pe1_body,
        grid=(GE,),
        in_specs=[
            pl.BlockSpec((BE, 80), lambda i: (i, 0)),
            pl.BlockSpec((BE, 80), lambda i: (i + GE, 0)),
        ]
        + [_full(w) for w in ws],
        out_specs=[
            pl.BlockSpec((NC, BE, H // 2), lambda i: (0, i, 0)),
            pl.BlockSpec((BE, H), lambda i: (i, 0)),
        ],
        out_shape=[
            jax.ShapeDtypeStruct((NC, E, H // 2), jnp.float32),
            jax.ShapeDtypeStruct((E, H), jnp.float32),
        ],
    )(xnf_g, xnf_g, *ws)


def _pe_call(xg, eh, w0t, b0, w1t, b1, g, beta):
    ws = (w0t, b0, w1t, b1, g, beta)
    return pl.pallas_call(
        _pe_body,
        grid=(GE,),
        in_specs=[
            pl.BlockSpec((BE, H), lambda i: (i, 0)),
            pl.BlockSpec((BE, H), lambda i: (i + GE, 0)),
            pl.BlockSpec((BE, H), lambda i: (i, 0)),
        ]
        + [_full(w) for w in ws],
        out_specs=[
            pl.BlockSpec((NC, BE, H // 2), lambda i: (0, i, 0)),
            pl.BlockSpec((BE, H), lambda i: (i, 0)),
        ],
        out_shape=[
            jax.ShapeDtypeStruct((NC, E, H // 2), jnp.float32),
            jax.ShapeDtypeStruct((E, H), jnp.float32),
        ],
    )(xg, xg, eh, *ws)


def _pn_body(a0_ref, a1_ref, xh_ref, w0_ref, b0_ref, w1_ref, b1_ref,
             g_ref, beta_ref, o_ref):
    xh = xh_ref[...]
    cat = jnp.concatenate([a0_ref[0], a1_ref[0], xh], axis=-1)
    h = jnp.maximum(_mm(cat, w0_ref[...]) + b0_ref[...], 0.0)
    y = _mm(h, w1_ref[...]) + b1_ref[...]
    o_ref[...] = xh + _ln(y, g_ref[...], beta_ref[...])


def _pn_call(aggr2, xh, w0t, b0, w1t, b1, g, beta):
    ws = (w0t, b0, w1t, b1, g, beta)
    return pl.pallas_call(
        _pn_body,
        grid=(GN,),
        in_specs=[
            pl.BlockSpec((1, BN, H // 2), lambda i: (0, i, 0)),
            pl.BlockSpec((1, BN, H // 2), lambda i: (1, i, 0)),
            pl.BlockSpec((BN, H), lambda i: (i, 0)),
        ]
        + [_full(w) for w in ws],
        out_specs=pl.BlockSpec((BN, H), lambda i: (i, 0)),
        out_shape=jax.ShapeDtypeStruct((N, H), jnp.float32),
    )(aggr2, aggr2, xh, *ws)


def _dec_body(xh_ref, w1_ref, b1_ref, w2_ref, b2_ref, o_ref):
    h = _mm(xh_ref[...], w1_ref[...]) + b1_ref[...]
    h = h * jax.nn.sigmoid(h)
    o_ref[...] = _mm(h, w2_ref[...]) + b2_ref[...]


def _dec_call(xh, w1t, b1, w2t, b2):
    ws = (w1t, b1, w2t, b2)
    return pl.pallas_call(
        _dec_body,
        grid=(GN,),
        in_specs=[pl.BlockSpec((BN, H), lambda i: (i, 0))]
        + [_full(w) for w in ws],
        out_specs=pl.BlockSpec((BN, 8), lambda i: (i, 0)),
        out_shape=jax.ShapeDtypeStruct((N, 8), jnp.float32),
    )(xh, *ws)


def kernel(world_pos, mesh_pos, phi, swelling_phi, swelling_phi_rate,
           node_type, mat_param, params, edge_index):
    f32 = jnp.float32
    mat = jnp.broadcast_to(mat_param[None, :].astype(f32), (N, 4))
    x = jnp.concatenate(
        [phi, swelling_phi, swelling_phi_rate, node_type, mat], axis=-1
    )
    nf = jnp.concatenate(
        [mesh_pos, world_pos, phi, jnp.zeros((N, 9), f32)], axis=-1
    )

    ei = edge_index.astype(jnp.int32)
    idx_flat = jnp.concatenate(
        [ei[0], ei[1], jnp.zeros((R_PAD - 2 * E,), jnp.int32)]
    ).reshape(G_CHUNKS_PAD, CH)
    sidx = ei[1].reshape(S_CHUNKS, CH)
    zrows = jnp.zeros((ZROWS, H // 2), f32)

    ne = params['node_enc']
    x_h = _mlp_call(x, ne['W0'].T, ne['b0'][None], ne['W1'].T, ne['b1'][None],
                    ne['g'][None], ne['beta'][None], BN)

    ee = params['edge_enc']
    w0t9 = ee['W0'].T
    w0p = jnp.concatenate(
        [w0t9[0:3], w0t9[4:7], w0t9[8:9], jnp.zeros((9, H), f32)], axis=0)
    w3 = w0t9[3:4].astype(jnp.bfloat16).astype(f32)
    w7 = w0t9[7:8].astype(jnp.bfloat16).astype(f32)
    ee_ws = (w0p, w3, w7, ee['b0'][None], ee['W1'].T, ee['b1'][None],
             ee['g'][None], ee['beta'][None])

    e_h = None
    msg2 = None
    for si, p in enumerate(params['procs']):
        em = p['edge_mlp']
        nm = p['node_mlp']
        pe_ws = (em['W0'].T, em['b0'][None], em['W1'].T, em['b1'][None],
                 em['g'][None], em['beta'][None])
        if si == 0:
            xnf_g = _GATHER80(jnp.concatenate([x_h, nf], axis=1), idx_flat)
            msg2, e_h = _pe1_call(xnf_g, ee_ws, pe_ws)
        else:
            xg = _GATHER64(x_h, idx_flat)
            msg2, e_h = _pe_call(xg, e_h, *pe_ws)
        aggr2 = _scatter_add(msg2, sidx, zrows)
        x_h = _pn_call(
            aggr2, x_h, nm['W0'].T, nm['b0'][None], nm['W1'].T, nm['b1'][None],
            nm['g'][None], nm['beta'][None],
        )

    d = params['dec']
    w2p = jnp.concatenate([d['W2'].T, jnp.zeros((H // 2, 5), f32)], axis=1)
    b2p = jnp.concatenate([d['b2'], jnp.zeros((5,), f32)])[None]
    dec = _dec_call(x_h, d['W1'].T, d['b1'][None], w2p, b2p)
    return dec[:, :3].reshape(1, N, 3)


# speedup vs baseline: 1.1811x; 1.0084x over previous
import functools

import jax
import jax.numpy as jnp
from jax import lax
from jax.experimental import pallas as pl
from jax.experimental.pallas import tpu as pltpu
from jax.experimental.pallas import tpu_sc as plsc

N = 50000
E = 800000
H = 64

NC = 2
NS = 16
NW = NC * NS

CH = 128

G_CHUNKS = (2 * E) // CH
G_CHUNKS_PAD = 12544
R_PAD = G_CHUNKS_PAD * CH
W_CHUNKS = G_CHUNKS_PAD // NW

S_CHUNKS = E // CH
SK = 4
S_MAIN_ITERS = 96
S_MAIN = S_MAIN_ITERS * SK
S_EXTRA_BASE = NS * S_MAIN
ZROWS = N // NS

BE = 8000
BN = 2000
GE = E // BE
GN = N // BN

_MESH = plsc.VectorSubcoreMesh(core_axis_name="c", subcore_axis_name="s")


def _make_gather(d, dtype, gk):
    iters = W_CHUNKS // gk
    assert W_CHUNKS % gk == 0
    @functools.partial(
        pl.kernel,
        out_type=jax.ShapeDtypeStruct((R_PAD, d), dtype),
        mesh=_MESH,
        scratch_types=[
            pltpu.VMEM((2, gk, CH), jnp.int32),
            pltpu.VMEM((2, gk * CH, d), dtype),
            pltpu.SemaphoreType.DMA,
            pltpu.SemaphoreType.DMA,
            pltpu.SemaphoreType.DMA,
        ],
        compiler_params=pltpu.CompilerParams(use_tc_tiling_on_sc=False),
    )
    def gather_k(table_hbm, idx_hbm, out_hbm, idx_v, rows_v, sem_g, sem_w0,
                 sem_w1):
        wid = lax.axis_index("s") * NC + lax.axis_index("c")
        base = wid * W_CHUNKS
        sem_w = (sem_w0, sem_w1)

        def fire(j, b):
            cid0 = base + j * gk
            pltpu.sync_copy(idx_hbm.at[pl.ds(cid0, gk)], idx_v.at[b])
            for k in range(gk):
                pltpu.async_copy(
                    table_hbm.at[idx_v.at[b].at[k]],
                    rows_v.at[b].at[pl.ds(k * CH, CH)],
                    sem_g,
                )

        def drain_gathers(b):
            for k in range(gk):
                pltpu.make_async_copy(
                    table_hbm.at[idx_v.at[b].at[k]],
                    rows_v.at[b].at[pl.ds(k * CH, CH)],
                    sem_g,
                ).wait()

        def writeback(j, b, sem):
            cid0 = base + j * gk
            pltpu.async_copy(rows_v.at[b], out_hbm.at[pl.ds(cid0 * CH, gk * CH)],
                             sem)

        def wait_writeback(b, sem):
            pltpu.make_async_copy(out_hbm.at[pl.ds(0, gk * CH)], rows_v.at[b],
                                  sem).wait()

        fire(0, 0)

        def step(j, carry):
            def phase(b_cur, b_prev):
                @pl.when(j >= 2)
                def _():
                    wait_writeback(b_cur, sem_w[b_cur])
                drain_gathers(b_prev)
                writeback(j - 1, b_prev, sem_w[b_prev])
                fire(j, b_cur)

            @pl.when(j % 2 == 0)
            def _():
                phase(0, 1)

            @pl.when(j % 2 == 1)
            def _():
                phase(1, 0)

            return carry

        lax.fori_loop(1, iters, step, 0)

        last = iters - 1
        bl = last % 2
        drain_gathers(bl)
        wait_writeback(1 - bl, sem_w[1 - bl])
        pltpu.sync_copy(rows_v.at[bl],
                        out_hbm.at[pl.ds((base + last * gk) * CH, gk * CH)])

    return gather_k


_GATHER80 = _make_gather(80, jnp.float32, 4)
_GATHER64 = _make_gather(64, jnp.float32, 7)


@functools.partial(
    pl.kernel,
    out_type=jax.ShapeDtypeStruct((NC, N, H // 2), jnp.float32),
    mesh=_MESH,
    scratch_types=[
        pltpu.VMEM((SK, CH), jnp.int32),
        pltpu.VMEM((SK * CH, H // 2), jnp.float32),
        pltpu.VMEM_SHARED((N, H // 2), jnp.float32),
    ],
    compiler_params=pltpu.CompilerParams(use_tc_tiling_on_sc=False),
)
def _scatter_add(msg_hbm, idx_hbm, zero_hbm, out_hbm, idx_v, msg_v, shared):
    c = lax.axis_index("c")
    t = lax.axis_index("s")
    pltpu.sync_copy(zero_hbm, shared.at[pl.ds(t * ZROWS, ZROWS)])
    plsc.subcore_barrier()

    def step(i, carry):
        cid0 = t * S_MAIN + i * SK
        pltpu.sync_copy(idx_hbm.at[pl.ds(cid0, SK)], idx_v)
        pltpu.sync_copy(msg_hbm.at[c, pl.ds(cid0 * CH, SK * CH)], msg_v)
        for k in range(SK):
            pltpu.sync_copy(
                msg_v.at[pl.ds(k * CH, CH)], shared.at[idx_v.at[k]], add=True
            )
        return carry

    lax.fori_loop(0, S_MAIN_ITERS, step, 0)

    extra_cnt = jnp.where(t < 10, 7, 6)
    extra_base = S_EXTRA_BASE + t * 6 + jnp.minimum(t, 10)

    def estep(j, carry):
        cid = extra_base + j
        pltpu.sync_copy(idx_hbm.at[pl.ds(cid, 1)], idx_v.at[pl.ds(0, 1)])
        pltpu.sync_copy(msg_hbm.at[c, pl.ds(cid * CH, CH)], msg_v.at[pl.ds(0, CH)])
        pltpu.sync_copy(msg_v.at[pl.ds(0, CH)], shared.at[idx_v.at[0]], add=True)
        return carry

    lax.fori_loop(0, extra_cnt, estep, 0)
    plsc.subcore_barrier()
    pltpu.sync_copy(
        shared.at[pl.ds(t * ZROWS, ZROWS)], out_hbm.at[c, pl.ds(t * ZROWS, ZROWS)]
    )


def _mm(a, b):
    return lax.dot_general(
        a, b, (((1,), (0,)), ((), ())),
        preferred_element_type=jnp.float32,
    )


def _ln(y, g, beta):
    mu = jnp.mean(y, axis=-1, keepdims=True)
    var = jnp.mean((y - mu) ** 2, axis=-1, keepdims=True)
    return (y - mu) / jnp.sqrt(var + 1e-5) * g + beta


def _full(x):
    return pl.BlockSpec(x.shape, lambda i: tuple(0 for _ in x.shape))


def _mlp_body(x_ref, w0_ref, b0_ref, w1_ref, b1_ref, g_ref, beta_ref, o_ref):
    h = jnp.maximum(_mm(x_ref[...], w0_ref[...]) + b0_ref[...], 0.0)
    y = _mm(h, w1_ref[...]) + b1_ref[...]
    o_ref[...] = _ln(y, g_ref[...], beta_ref[...])


def _mlp_call(x, w0t, b0, w1t, b1, g, beta, bm):
    n = x.shape[0]
    ws = (w0t, b0, w1t, b1, g, beta)
    return pl.pallas_call(
        _mlp_body,
        grid=(n // bm,),
        in_specs=[pl.BlockSpec((bm, x.shape[1]), lambda i: (i, 0))]
        + [_full(w) for w in ws],
        out_specs=pl.BlockSpec((bm, H), lambda i: (i, 0)),
        out_shape=jax.ShapeDtypeStruct((n, H), jnp.float32),
    )(x, *ws)


def _edge_enc_body(nfs_ref, nfr_ref, w0_ref, b0_ref, w1_ref, b1_ref, g_ref,
                   beta_ref, o_ref):
    d = nfs_ref[...] - nfr_ref[...]
    dist = jnp.sqrt(d[:, 0:1] ** 2 + d[:, 1:2] ** 2 + d[:, 2:3] ** 2)
    dist_w = jnp.sqrt(d[:, 3:4] ** 2 + d[:, 4:5] ** 2 + d[:, 5:6] ** 2)
    e = jnp.concatenate(
        [d[:, 0:3], dist, d[:, 3:6], dist_w, d[:, 6:7],
         jnp.zeros_like(d[:, 0:7])], axis=-1)
    h = jnp.maximum(_mm(e, w0_ref[...]) + b0_ref[...], 0.0)
    y = _mm(h, w1_ref[...]) + b1_ref[...]
    o_ref[...] = _ln(y, g_ref[...], beta_ref[...])


def _edge_enc_call(nf_g, w0t, b0, w1t, b1, g, beta):
    ws = (w0t, b0, w1t, b1, g, beta)
    return pl.pallas_call(
        _edge_enc_body,
        grid=(GE,),
        in_specs=[
            pl.BlockSpec((BE, 16), lambda i: (i, 0)),
            pl.BlockSpec((BE, 16), lambda i: (i + GE, 0)),
        ]
        + [_full(w) for w in ws],
        out_specs=pl.BlockSpec((BE, H), lambda i: (i, 0)),
        out_shape=jax.ShapeDtypeStruct((E, H), jnp.float32),
    )(nf_g, nf_g, *ws)


def _pe_math(xs, xr, eh, w0, b0, w1, b1, g_, beta):
    h1 = jnp.maximum(_mm(jnp.concatenate([xr, xs, eh], axis=-1), w0) + b0, 0.0)
    m = _ln(_mm(h1, w1) + b1, g_, beta)
    h2 = jnp.maximum(_mm(jnp.concatenate([xs, xr, eh], axis=-1), w0) + b0, 0.0)
    ne = _ln(_mm(h2, w1) + b1, g_, beta)
    return m, ne


def _ee_math(nfs, nfr, w0p, w3, w7, b0, w1, b1, g_, beta):
    d = nfs - nfr
    dist = jnp.sqrt(d[:, 0:1] ** 2 + d[:, 1:2] ** 2 + d[:, 2:3] ** 2)
    dist_w = jnp.sqrt(d[:, 3:4] ** 2 + d[:, 4:5] ** 2 + d[:, 5:6] ** 2)
    db = dist.astype(jnp.bfloat16).astype(jnp.float32)
    dwb = dist_w.astype(jnp.bfloat16).astype(jnp.float32)
    h = jnp.maximum(_mm(d, w0p) + db * w3 + dwb * w7 + b0, 0.0)
    y = _mm(h, w1) + b1
    return _ln(y, g_, beta)


def _pe_body(xgs_ref, xgr_ref, eh_ref, w0_ref, b0_ref,
             w1_ref, b1_ref, g_ref, beta_ref, msg_ref, eo_ref):
    eh = eh_ref[...]
    m, ne = _pe_math(xgs_ref[...], xgr_ref[...], eh, w0_ref[...], b0_ref[...],
                     w1_ref[...], b1_ref[...], g_ref[...], beta_ref[...])
    eo_ref[...] = eh + ne
    msg_ref[0] = m[:, 0:32]
    msg_ref[1] = m[:, 32:64]


def _pe1_body(gs_ref, gr_ref,
              ew0_ref, ew3_ref, ew7_ref, eb0_ref, ew1_ref, eb1_ref, eg_ref,
              ebeta_ref,
              w0_ref, b0_ref, w1_ref, b1_ref, g_ref, beta_ref,
              msg_ref, eo_ref):
    gs = gs_ref[...]
    gr = gr_ref[...]
    eh = _ee_math(gs[:, 64:80], gr[:, 64:80], ew0_ref[...], ew3_ref[...],
                  ew7_ref[...], eb0_ref[...], ew1_ref[...], eb1_ref[...],
                  eg_ref[...], ebeta_ref[...])
    m, ne = _pe_math(gs[:, 0:64], gr[:, 0:64], eh, w0_ref[...], b0_ref[...],
                     w1_ref[...], b1_ref[...], g_ref[...], beta_ref[...])
    eo_ref[...] = eh + ne
    msg_ref[0] = m[:, 0:32]
    msg_ref[1] = m[:, 32:64]


def _pe1_call(xnf_g, ee_ws, pe_ws):
    ws = tuple(ee_ws) + tuple(pe_ws)
    return pl.pallas_call(
        _pe1_body,
        grid=(GE,),
        in_specs=[
            pl.BlockSpec((BE, 80), lambda i: (i, 0)),
            pl.BlockSpec((BE, 80), lambda i: (i + GE, 0)),
        ]
        + [_full(w) for w in ws],
        out_specs=[
            pl.BlockSpec((NC, BE, H // 2), lambda i: (0, i, 0)),
            pl.BlockSpec((BE, H), lambda i: (i, 0)),
        ],
        out_shape=[
            jax.ShapeDtypeStruct((NC, E, H // 2), jnp.float32),
            jax.ShapeDtypeStruct((E, H), jnp.float32),
        ],
    )(xnf_g, xnf_g, *ws)


def _pe_call(xg, eh, w0t, b0, w1t, b1, g, beta):
    ws = (w0t, b0, w1t, b1, g, beta)
    return pl.pallas_call(
        _pe_body,
        grid=(GE,),
        in_specs=[
            pl.BlockSpec((BE, H), lambda i: (i, 0)),
            pl.BlockSpec((BE, H), lambda i: (i + GE, 0)),
            pl.BlockSpec((BE, H), lambda i: (i, 0)),
        ]
        + [_full(w) for w in ws],
        out_specs=[
            pl.BlockSpec((NC, BE, H // 2), lambda i: (0, i, 0)),
            pl.BlockSpec((BE, H), lambda i: (i, 0)),
        ],
        out_shape=[
            jax.ShapeDtypeStruct((NC, E, H // 2), jnp.float32),
            jax.ShapeDtypeStruct((E, H), jnp.float32),
        ],
    )(xg, xg, eh, *ws)


def _pn_body(a0_ref, a1_ref, xh_ref, w0_ref, b0_ref, w1_ref, b1_ref,
             g_ref, beta_ref, o_ref):
    xh = xh_ref[...]
    cat = jnp.concatenate([a0_ref[0], a1_ref[0], xh], axis=-1)
    h = jnp.maximum(_mm(cat, w0_ref[...]) + b0_ref[...], 0.0)
    y = _mm(h, w1_ref[...]) + b1_ref[...]
    o_ref[...] = xh + _ln(y, g_ref[...], beta_ref[...])


def _pn_call(aggr2, xh, w0t, b0, w1t, b1, g, beta):
    ws = (w0t, b0, w1t, b1, g, beta)
    return pl.pallas_call(
        _pn_body,
        grid=(GN,),
        in_specs=[
            pl.BlockSpec((1, BN, H // 2), lambda i: (0, i, 0)),
            pl.BlockSpec((1, BN, H // 2), lambda i: (1, i, 0)),
            pl.BlockSpec((BN, H), lambda i: (i, 0)),
        ]
        + [_full(w) for w in ws],
        out_specs=pl.BlockSpec((BN, H), lambda i: (i, 0)),
        out_shape=jax.ShapeDtypeStruct((N, H), jnp.float32),
    )(aggr2, aggr2, xh, *ws)


def _dec_body(xh_ref, w1_ref, b1_ref, w2_ref, b2_ref, o_ref):
    h = _mm(xh_ref[...], w1_ref[...]) + b1_ref[...]
    h = h * jax.nn.sigmoid(h)
    o_ref[...] = _mm(h, w2_ref[...]) + b2_ref[...]


def _dec_call(xh, w1t, b1, w2t, b2):
    ws = (w1t, b1, w2t, b2)
    return pl.pallas_call(
        _dec_body,
        grid=(GN,),
        in_specs=[pl.BlockSpec((BN, H), lambda i: (i, 0))]
        + [_full(w) for w in ws],
        out_specs=pl.BlockSpec((BN, 8), lambda i: (i, 0)),
        out_shape=jax.ShapeDtypeStruct((N, 8), jnp.float32),
    )(xh, *ws)


def kernel(world_pos, mesh_pos, phi, swelling_phi, swelling_phi_rate,
           node_type, mat_param, params, edge_index):
    f32 = jnp.float32
    mat = jnp.broadcast_to(mat_param[None, :].astype(f32), (N, 4))
    x = jnp.concatenate(
        [phi, swelling_phi, swelling_phi_rate, node_type, mat], axis=-1
    )
    nf = jnp.concatenate(
        [mesh_pos, world_pos, phi, jnp.zeros((N, 9), f32)], axis=-1
    )

    ei = edge_index.astype(jnp.int32)
    idx_flat = jnp.concatenate(
        [ei[0], ei[1], jnp.zeros((R_PAD - 2 * E,), jnp.int32)]
    ).reshape(G_CHUNKS_PAD, CH)
    sidx = ei[1].reshape(S_CHUNKS, CH)
    zrows = jnp.zeros((ZROWS, H // 2), f32)

    ne = params['node_enc']
    x_h = _mlp_call(x, ne['W0'].T, ne['b0'][None], ne['W1'].T, ne['b1'][None],
                    ne['g'][None], ne['beta'][None], BN)

    ee = params['edge_enc']
    w0t9 = ee['W0'].T
    w0p = jnp.concatenate(
        [w0t9[0:3], w0t9[4:7], w0t9[8:9], jnp.zeros((9, H), f32)], axis=0)
    w3 = w0t9[3:4].astype(jnp.bfloat16).astype(f32)
    w7 = w0t9[7:8].astype(jnp.bfloat16).astype(f32)
    ee_ws = (w0p, w3, w7, ee['b0'][None], ee['W1'].T, ee['b1'][None],
             ee['g'][None], ee['beta'][None])

    e_h = None
    msg2 = None
    for si, p in enumerate(params['procs']):
        em = p['edge_mlp']
        nm = p['node_mlp']
        pe_ws = (em['W0'].T, em['b0'][None], em['W1'].T, em['b1'][None],
                 em['g'][None], em['beta'][None])
        if si == 0:
            xnf_g = _GATHER80(jnp.concatenate([x_h, nf], axis=1), idx_flat)
            msg2, e_h = _pe1_call(xnf_g, ee_ws, pe_ws)
        else:
            xg = _GATHER64(x_h, idx_flat)
            msg2, e_h = _pe_call(xg, e_h, *pe_ws)
        aggr2 = _scatter_add(msg2, sidx, zrows)
        x_h = _pn_call(
            aggr2, x_h, nm['W0'].T, nm['b0'][None], nm['W1'].T, nm['b1'][None],
            nm['g'][None], nm['beta'][None],
        )

    d = params['dec']
    w2p = jnp.concatenate([d['W2'].T, jnp.zeros((H // 2, 5), f32)], axis=1)
    b2p = jnp.concatenate([d['b2'], jnp.zeros((5,), f32)])[None]
    dec = _dec_call(x_h, d['W1'].T, d['b1'][None], w2p, b2p)
    return dec[:, :3].reshape(1, N, 3)
